# baseline (speedup 1.0000x reference)
import jax
import jax.numpy as jnp
from jax.experimental import pallas as pl
from jax.experimental.pallas import tpu as pltpu

NUM_GROUPS = 4
TOKENS_PER_GROUP = 8192
HIDDEN = 4096
NUM_EXPERTS = 64
TILE = 512


def _router_body(x_ref, w_ref, b_ref, logits_ref, probs_ref, zsum_ref):
    logits = jnp.dot(x_ref[...], w_ref[...],
                     preferred_element_type=jnp.float32)
    logits = logits + b_ref[...]
    logits_ref[...] = logits
    m = jnp.max(logits, axis=-1, keepdims=True)
    e = jnp.exp(logits - m)
    s = jnp.sum(e, axis=-1, keepdims=True)
    probs_ref[...] = e / s
    log_z = m + jnp.log(s)
    part = jnp.sum(log_z * log_z)

    @pl.when(pl.program_id(0) == 0)
    def _():
        zsum_ref[0, 0] = 0.0

    zsum_ref[0, 0] += part


def kernel(token_inputs, W, b, expert_capacity):
    n_tokens = NUM_GROUPS * TOKENS_PER_GROUP
    x = token_inputs.reshape(n_tokens, HIDDEN)
    b2 = b.reshape(1, NUM_EXPERTS)
    grid = (n_tokens // TILE,)
    logits, probs, zsum = pl.pallas_call(
        _router_body,
        grid=grid,
        in_specs=[
            pl.BlockSpec((TILE, HIDDEN), lambda i: (i, 0)),
            pl.BlockSpec((HIDDEN, NUM_EXPERTS), lambda i: (0, 0)),
            pl.BlockSpec((1, NUM_EXPERTS), lambda i: (0, 0)),
        ],
        out_specs=[
            pl.BlockSpec((TILE, NUM_EXPERTS), lambda i: (i, 0)),
            pl.BlockSpec((TILE, NUM_EXPERTS), lambda i: (i, 0)),
            pl.BlockSpec(block_shape=(1, 1), index_map=lambda i: (0, 0),
                         memory_space=pltpu.MemorySpace.SMEM),
        ],
        out_shape=[
            jax.ShapeDtypeStruct((n_tokens, NUM_EXPERTS), jnp.float32),
            jax.ShapeDtypeStruct((n_tokens, NUM_EXPERTS), jnp.float32),
            jax.ShapeDtypeStruct((1, 1), jnp.float32),
        ],
    )(x, W, b2)
    z_loss = zsum[0, 0] / n_tokens
    shape3 = (NUM_GROUPS, TOKENS_PER_GROUP, NUM_EXPERTS)
    return (probs.reshape(shape3), logits.reshape(shape3), z_loss)


# speedup vs baseline: 1.0267x; 1.0267x over previous
import jax
import jax.numpy as jnp
from jax.experimental import pallas as pl
from jax.experimental.pallas import tpu as pltpu

NUM_GROUPS = 4
TOKENS_PER_GROUP = 8192
HIDDEN = 4096
NUM_EXPERTS = 64
TILE = 1024


def _router_body(x_ref, w_ref, b_ref, logits_ref, probs_ref, zsum_ref):
    logits = jnp.dot(x_ref[...], w_ref[...],
                     preferred_element_type=jnp.float32)
    logits = logits + b_ref[...]
    logits_ref[...] = logits
    m = jnp.max(logits, axis=-1, keepdims=True)
    e = jnp.exp(logits - m)
    s = jnp.sum(e, axis=-1, keepdims=True)
    probs_ref[...] = e / s
    log_z = m + jnp.log(s)
    part = jnp.sum(log_z * log_z)

    @pl.when(pl.program_id(0) == 0)
    def _():
        zsum_ref[0, 0] = 0.0

    zsum_ref[0, 0] += part


def kernel(token_inputs, W, b, expert_capacity):
    n_tokens = NUM_GROUPS * TOKENS_PER_GROUP
    x = token_inputs.reshape(n_tokens, HIDDEN)
    b2 = b.reshape(1, NUM_EXPERTS)
    grid = (n_tokens // TILE,)
    logits, probs, zsum = pl.pallas_call(
        _router_body,
        grid=grid,
        in_specs=[
            pl.BlockSpec((TILE, HIDDEN), lambda i: (i, 0)),
            pl.BlockSpec((HIDDEN, NUM_EXPERTS), lambda i: (0, 0)),
            pl.BlockSpec((1, NUM_EXPERTS), lambda i: (0, 0)),
        ],
        out_specs=[
            pl.BlockSpec((TILE, NUM_EXPERTS), lambda i: (i, 0)),
            pl.BlockSpec((TILE, NUM_EXPERTS), lambda i: (i, 0)),
            pl.BlockSpec(block_shape=(1, 1), index_map=lambda i: (0, 0),
                         memory_space=pltpu.MemorySpace.SMEM),
        ],
        out_shape=[
            jax.ShapeDtypeStruct((n_tokens, NUM_EXPERTS), jnp.float32),
            jax.ShapeDtypeStruct((n_tokens, NUM_EXPERTS), jnp.float32),
            jax.ShapeDtypeStruct((1, 1), jnp.float32),
        ],
    )(x, W, b2)
    z_loss = zsum[0, 0] / n_tokens
    shape3 = (NUM_GROUPS, TOKENS_PER_GROUP, NUM_EXPERTS)
    return (probs.reshape(shape3), logits.reshape(shape3), z_loss)


# speedup vs baseline: 1.0296x; 1.0028x over previous
import jax
import jax.numpy as jnp
from jax.experimental import pallas as pl
from jax.experimental.pallas import tpu as pltpu

NUM_GROUPS = 4
TOKENS_PER_GROUP = 8192
HIDDEN = 4096
NUM_EXPERTS = 64
TILE = 1024


def _router_body(x_ref, w_ref, b_ref, logits_ref, probs_ref, zpart_ref):
    logits = jnp.dot(x_ref[...], w_ref[...],
                     preferred_element_type=jnp.float32)
    logits = logits + b_ref[...]
    logits_ref[...] = logits
    m = jnp.max(logits, axis=-1, keepdims=True)
    e = jnp.exp(logits - m)
    s = jnp.sum(e, axis=-1, keepdims=True)
    probs_ref[...] = e / s
    log_z = m + jnp.log(s)
    zpart_ref[0, 0, 0] = jnp.sum(log_z * log_z)


def kernel(token_inputs, W, b, expert_capacity):
    n_tokens = NUM_GROUPS * TOKENS_PER_GROUP
    n_tiles = n_tokens // TILE
    x = token_inputs.reshape(n_tokens, HIDDEN)
    b2 = b.reshape(1, NUM_EXPERTS)
    logits, probs, zparts = pl.pallas_call(
        _router_body,
        grid=(n_tiles,),
        in_specs=[
            pl.BlockSpec((TILE, HIDDEN), lambda i: (i, 0)),
            pl.BlockSpec((HIDDEN, NUM_EXPERTS), lambda i: (0, 0)),
            pl.BlockSpec((1, NUM_EXPERTS), lambda i: (0, 0)),
        ],
        out_specs=[
            pl.BlockSpec((TILE, NUM_EXPERTS), lambda i: (i, 0)),
            pl.BlockSpec((TILE, NUM_EXPERTS), lambda i: (i, 0)),
            pl.BlockSpec(block_shape=(1, 1, 1), index_map=lambda i: (i, 0, 0),
                         memory_space=pltpu.MemorySpace.SMEM),
        ],
        out_shape=[
            jax.ShapeDtypeStruct((n_tokens, NUM_EXPERTS), jnp.float32),
            jax.ShapeDtypeStruct((n_tokens, NUM_EXPERTS), jnp.float32),
            jax.ShapeDtypeStruct((n_tiles, 1, 1), jnp.float32),
        ],
        compiler_params=pltpu.CompilerParams(
            dimension_semantics=("parallel",),
        ),
    )(x, W, b2)
    z_loss = jnp.sum(zparts) / n_tokens
    shape3 = (NUM_GROUPS, TOKENS_PER_GROUP, NUM_EXPERTS)
    return (probs.reshape(shape3), logits.reshape(shape3), z_loss)


# speedup vs baseline: 1.0715x; 1.0407x over previous
import jax
import jax.numpy as jnp
from jax.experimental import pallas as pl
from jax.experimental.pallas import tpu as pltpu

NUM_GROUPS = 4
TOKENS_PER_GROUP = 8192
HIDDEN = 4096
NUM_EXPERTS = 64
TILE = 1024
TILES_PER_GROUP = TOKENS_PER_GROUP // TILE


def _router_body(x_ref, w_ref, b_ref, logits_ref, probs_ref, zpart_ref):
    x = x_ref[0]
    logits = jnp.dot(x, w_ref[...], preferred_element_type=jnp.float32)
    logits = logits + b_ref[...]
    logits_ref[0] = logits
    m = jnp.max(logits, axis=-1, keepdims=True)
    e = jnp.exp(logits - m)
    s = jnp.sum(e, axis=-1, keepdims=True)
    probs_ref[0] = e / s
    log_z = m + jnp.log(s)
    zpart_ref[0, 0, 0] = jnp.sum(log_z * log_z)


def kernel(token_inputs, W, b, expert_capacity):
    n_tokens = NUM_GROUPS * TOKENS_PER_GROUP
    n_tiles = n_tokens // TILE
    b2 = b.reshape(1, NUM_EXPERTS)
    shape3 = (NUM_GROUPS, TOKENS_PER_GROUP, NUM_EXPERTS)
    logits, probs, zparts = pl.pallas_call(
        _router_body,
        grid=(n_tiles,),
        in_specs=[
            pl.BlockSpec((1, TILE, HIDDEN),
                         lambda i: (i // TILES_PER_GROUP,
                                    i % TILES_PER_GROUP, 0)),
            pl.BlockSpec((HIDDEN, NUM_EXPERTS), lambda i: (0, 0)),
            pl.BlockSpec((1, NUM_EXPERTS), lambda i: (0, 0)),
        ],
        out_specs=[
            pl.BlockSpec((1, TILE, NUM_EXPERTS),
                         lambda i: (i // TILES_PER_GROUP,
                                    i % TILES_PER_GROUP, 0)),
            pl.BlockSpec((1, TILE, NUM_EXPERTS),
                         lambda i: (i // TILES_PER_GROUP,
                                    i % TILES_PER_GROUP, 0)),
            pl.BlockSpec(block_shape=(1, 1, 1), index_map=lambda i: (i, 0, 0),
                         memory_space=pltpu.MemorySpace.SMEM),
        ],
        out_shape=[
            jax.ShapeDtypeStruct(shape3, jnp.float32),
            jax.ShapeDtypeStruct(shape3, jnp.float32),
            jax.ShapeDtypeStruct((n_tiles, 1, 1), jnp.float32),
        ],
        compiler_params=pltpu.CompilerParams(
            dimension_semantics=("parallel",),
        ),
    )(token_inputs, W, b2)
    z_loss = jnp.sum(zparts) / n_tokens
    return (probs, logits, z_loss)
